# baseline (speedup 1.0000x reference)
import functools

import jax
import jax.numpy as jnp
from jax import lax
from jax.experimental import pallas as pl
from jax.experimental.pallas import tpu as pltpu
from jax.experimental.pallas import tpu_sc as plsc

_N = 10000
_E = 320000
_DIN = 128
_DH = 128
_F = 64
_KHOP = 10
_TEMP = 1.5
_EPS = 1e-5

_NC, _NS, _L = 2, 16, 16
_NW = _NC * _NS
_NPAD = 10240
_RPT = _NPAD // _NW
_RPS = _NPAD // _NS
_RC = 64
_C = 128
_ET = _E + _N
_CHUNKS = 81
_EPT = _CHUNKS * _C
_ETPAD = _NW * _EPT
_TRASH = _N

_mesh = plsc.VectorSubcoreMesh(core_axis_name="c", subcore_axis_name="s")
_f32 = jnp.float32


def _gelu(t):
    return 0.5 * t * (1.0 + lax.erf(t * 0.7071067811865476))


def _mlp_body(x_ref, w1, b1, w2, b2, w3, b3, w4, b4, o_ref):
    xb = x_ref[...]
    x0 = _gelu(jnp.dot(xb, w1[...], preferred_element_type=_f32) + b1[...])
    x1 = _gelu(jnp.dot(x0, w2[...], preferred_element_type=_f32) + b2[...] + x0)
    x2 = _gelu(jnp.dot(x1, w3[...], preferred_element_type=_f32) + b3[...] + x1)
    o_ref[...] = jnp.dot(x2, w4[...], preferred_element_type=_f32) + b4[...]


_BM = 512


def _tc_mlp(xp, w1, b1, w2, b2, w3, b3, w4, b4):
    def _fixed(r, c):
        return pl.BlockSpec((r, c), lambda i: (0, 0))

    return pl.pallas_call(
        _mlp_body,
        grid=(_NPAD // _BM,),
        in_specs=[
            pl.BlockSpec((_BM, _DIN), lambda i: (i, 0)),
            _fixed(_DIN, _DH), _fixed(1, _DH),
            _fixed(_DH, _DH), _fixed(1, _DH),
            _fixed(_DH, _DH), _fixed(1, _DH),
            _fixed(_DH, _F), _fixed(1, _F),
        ],
        out_specs=pl.BlockSpec((_BM, _F), lambda i: (i, 0)),
        out_shape=jax.ShapeDtypeStruct((_NPAD, _F), _f32),
    )(xp, w1, b1, w2, b2, w3, b3, w4, b4)


@functools.partial(
    pl.kernel, mesh=_mesh,
    compiler_params=pltpu.CompilerParams(use_tc_tiling_on_sc=False),
    out_type=jax.ShapeDtypeStruct((_NC, _NPAD, _L), _f32),
    scratch_types=[
        pltpu.VMEM((_CHUNKS, _C), jnp.int32),
        pltpu.VMEM((_C, _L), _f32),
        pltpu.VMEM((_C, _L), _f32),
        pltpu.VMEM_SHARED((_NPAD, _L), _f32),
    ],
)
def _sc_deg(row_hbm, out_hbm, row_v, ones_v, zbuf, acc):
    c = lax.axis_index("c")
    s = lax.axis_index("s")
    wid = s * _NC + c

    def fill(j, _):
        ones_v[j, :] = jnp.full((_L,), 1.0, _f32)
        zbuf[j, :] = jnp.zeros((_L,), _f32)
        return 0

    lax.fori_loop(0, _C, fill, 0)
    for t in range(_RPS // _C):
        pltpu.sync_copy(zbuf, acc.at[pl.ds(s * _RPS + t * _C, _C)])
    pltpu.sync_copy(row_hbm.at[wid], row_v)
    plsc.subcore_barrier()

    def body(j, _):
        pltpu.sync_copy(ones_v, acc.at[row_v.at[j]], add=True)
        return 0

    lax.fori_loop(0, _CHUNKS, body, 0)
    plsc.subcore_barrier()
    pltpu.sync_copy(acc.at[pl.ds(s * _RPS, _RPS)],
                    out_hbm.at[c, pl.ds(s * _RPS, _RPS)])


@functools.partial(
    pl.kernel, mesh=_mesh,
    compiler_params=pltpu.CompilerParams(use_tc_tiling_on_sc=False),
    out_type=jax.ShapeDtypeStruct((_NC, _NPAD, _F), _f32),
    scratch_types=[
        pltpu.VMEM((_CHUNKS, _C), jnp.int32),
        pltpu.VMEM((_CHUNKS, _C), jnp.int32),
        pltpu.VMEM((_C, _F), _f32),
        pltpu.VMEM((_C, _F), _f32),
        pltpu.VMEM((_C, _F), _f32),
        pltpu.VMEM_SHARED((_NPAD, _F), _f32),
        pltpu.SemaphoreType.DMA,
        pltpu.SemaphoreType.DMA,
        pltpu.SemaphoreType.DMA,
    ],
)
def _sc_hop(w_hbm, col_hbm, row_hbm, out_hbm,
            col_v, row_v, buf0, buf1, buf2, acc, g0, g1, g2):
    c = lax.axis_index("c")
    s = lax.axis_index("s")
    wid = s * _NC + c
    bufs = (buf0, buf1, buf2)
    gsems = (g0, g1, g2)

    def zfill(j, _):
        for fb in range(_F // _L):
            buf0[j, pl.ds(fb * _L, _L)] = jnp.zeros((_L,), _f32)
        return 0

    lax.fori_loop(0, _C, zfill, 0)
    for t in range(_RPS // _C):
        pltpu.sync_copy(buf0, acc.at[pl.ds(s * _RPS + t * _C, _C)])
    pltpu.sync_copy(col_hbm.at[wid], col_v)
    pltpu.sync_copy(row_hbm.at[wid], row_v)
    plsc.subcore_barrier()

    for b in range(3):
        pltpu.async_copy(w_hbm.at[col_v.at[b]], bufs[b], gsems[b])

    def body(i, _):
        for b in range(3):
            j = 3 * i + b
            pltpu.make_async_copy(w_hbm.at[col_v.at[j]], bufs[b],
                                  gsems[b]).wait()
            pltpu.sync_copy(bufs[b], acc.at[row_v.at[j]], add=True)
            pltpu.async_copy(w_hbm.at[col_v.at[j + 3]], bufs[b], gsems[b])
        return 0

    lax.fori_loop(0, _CHUNKS // 3 - 1, body, 0)
    for b in range(3):
        j = _CHUNKS - 3 + b
        pltpu.make_async_copy(w_hbm.at[col_v.at[j]], bufs[b], gsems[b]).wait()
        pltpu.sync_copy(bufs[b], acc.at[row_v.at[j]], add=True)
    plsc.subcore_barrier()
    pltpu.sync_copy(acc.at[pl.ds(s * _RPS, _RPS)],
                    out_hbm.at[c, pl.ds(s * _RPS, _RPS)])


_BC = 2048


def _init_body(d0, d1, h_ref, a_ref, dinv_ref, w_ref, xp_ref):
    deg = jnp.maximum(d0[...][:, :1] + d1[...][:, :1], 1.0)
    dv = lax.rsqrt(deg)
    hb = h_ref[...]
    dinv_ref[...] = dv
    w_ref[...] = dv * hb
    xp_ref[...] = a_ref[...] * hb


def _tc_init(degp0, degp1, h, a):
    return pl.pallas_call(
        _init_body,
        grid=(_NPAD // _BC,),
        in_specs=[
            pl.BlockSpec((_BC, _L), lambda i: (i, 0)),
            pl.BlockSpec((_BC, _L), lambda i: (i, 0)),
            pl.BlockSpec((_BC, _F), lambda i: (i, 0)),
            pl.BlockSpec((1, 1), lambda i: (0, 0)),
        ],
        out_specs=[
            pl.BlockSpec((_BC, 1), lambda i: (i, 0)),
            pl.BlockSpec((_BC, _F), lambda i: (i, 0)),
            pl.BlockSpec((_BC, _F), lambda i: (i, 0)),
        ],
        out_shape=[
            jax.ShapeDtypeStruct((_NPAD, 1), _f32),
            jax.ShapeDtypeStruct((_NPAD, _F), _f32),
            jax.ShapeDtypeStruct((_NPAD, _F), _f32),
        ],
    )(degp0, degp1, h, a)


def _combine_body(p0, p1, dinv_ref, xp_in, a_ref, w_ref, xp_ref):
    sb = p0[...] + p1[...]
    dv = dinv_ref[...]
    dsv = dv * sb
    w_ref[...] = dv * dsv
    xp_ref[...] = xp_in[...] + a_ref[...] * dsv


def _tc_combine(p0, p1, dinv, xp, a):
    return pl.pallas_call(
        _combine_body,
        grid=(_NPAD // _BC,),
        in_specs=[
            pl.BlockSpec((_BC, _F), lambda i: (i, 0)),
            pl.BlockSpec((_BC, _F), lambda i: (i, 0)),
            pl.BlockSpec((_BC, 1), lambda i: (i, 0)),
            pl.BlockSpec((_BC, _F), lambda i: (i, 0)),
            pl.BlockSpec((1, 1), lambda i: (0, 0)),
        ],
        out_specs=[
            pl.BlockSpec((_BC, _F), lambda i: (i, 0)),
            pl.BlockSpec((_BC, _F), lambda i: (i, 0)),
        ],
        out_shape=[
            jax.ShapeDtypeStruct((_NPAD, _F), _f32),
            jax.ShapeDtypeStruct((_NPAD, _F), _f32),
        ],
    )(p0, p1, dinv, xp, a)


def kernel(x, edge_index, W1, b1, W2, b2, W3, b3, W4, b4,
           g1, be1, m1, v1, g2, be2, m2, v2, g3, be3, m3, v3, alpha):
    s1 = g1 * lax.rsqrt(v1 + _EPS)
    s2 = g2 * lax.rsqrt(v2 + _EPS)
    s3 = g3 * lax.rsqrt(v3 + _EPS)
    w1f = W1 * s1[None, :]
    w2f = W2 * s2[None, :]
    w3f = W3 * s3[None, :]
    b1f = (b1 * s1 + be1 - m1 * s1).reshape(1, _DH)
    b2f = (b2 * s2 + be2 - m2 * s2).reshape(1, _DH)
    b3f = (b3 * s3 + be3 - m3 * s3).reshape(1, _DH)

    xp = jnp.pad(x, ((0, _NPAD - _N), (0, 0)))
    h = _tc_mlp(xp, w1f, b1f, w2f, b2f, w3f, b3f, W4, b4.reshape(1, _F))

    loops = jnp.arange(_N, dtype=edge_index.dtype)
    row = jnp.concatenate([edge_index[0], loops,
                           jnp.full((_ETPAD - _ET,), _TRASH, edge_index.dtype)])
    col = jnp.concatenate([edge_index[1], loops,
                           jnp.zeros((_ETPAD - _ET,), edge_index.dtype)])
    row3 = row.reshape(_NW, _CHUNKS, _C)
    col3 = col.reshape(_NW, _CHUNKS, _C)

    avecs = (alpha / _TEMP).reshape(_KHOP + 1, 1)

    degp = _sc_deg(row3)
    dinv, w, xprop = _tc_init(degp[0], degp[1], h, avecs[0:1])
    for k in range(1, _KHOP + 1):
        p = _sc_hop(w, col3, row3)
        w, xprop = _tc_combine(p[0], p[1], dinv, xprop, avecs[k:k + 1])
    return xprop[:_N]
